# Initial kernel scaffold; baseline (speedup 1.0000x reference)
import jax
import jax.numpy as jnp
from jax.experimental import pallas as pl


def kernel(Xw, Xs, E, Erev, W1, b1, W2, b2, Wg1, bg1, Wg2, bg2, W3, b3, W4, b4):
    raise NotImplementedError("write your pallas kernel here")


# speedup vs baseline: 2.7008x; 2.7008x over previous
import functools

import jax
import jax.numpy as jnp
from jax import lax
from jax.experimental import pallas as pl
from jax.experimental.pallas import tpu as pltpu
from jax.experimental.pallas import tpu_sc as plsc

_NW = 10000
_NS = 10000
_NE = 160000
_D = 256
_HALF = 128
_FW = 144
_NTILES = 16
_EPT = _NE // _NTILES
_CHUNK = 80
_NCHUNK = _EPT // _CHUNK
_RPT = _NW // _NTILES

_f32 = jnp.float32


def _sc_aggregate(tabs, src2, dst, zeros):
    mesh = plsc.VectorSubcoreMesh(core_axis_name="c", subcore_axis_name="s")

    @functools.partial(
        pl.kernel,
        mesh=mesh,
        out_type=jax.ShapeDtypeStruct((2, 2, _NW, _FW), _f32),
        scratch_types=[pltpu.VMEM((_CHUNK,), jnp.int32),
                       pltpu.VMEM((_CHUNK,), jnp.int32),
                       pltpu.VMEM((_CHUNK, _FW), _f32),
                       pltpu.VMEM_SHARED((_NW, _FW), _f32),
                       pltpu.SemaphoreType.DMA],
        compiler_params=pltpu.CompilerParams(use_tc_tiling_on_sc=False),
    )
    def body(tabs_h, src2_h, dst_h, zro_h, out_h, srcv, dstv, rows, acc, sem):
        c = lax.axis_index("c")
        s = lax.axis_index("s")
        rbase = s * _RPT
        ebase = s * _EPT

        for t in range(2):
            pltpu.sync_copy(zro_h, acc.at[pl.ds(rbase, _RPT)])
            plsc.subcore_barrier()

            def chunk(j, carry):
                off = ebase + j * _CHUNK
                pltpu.sync_copy(src2_h.at[t, c, pl.ds(off, _CHUNK)], srcv)
                pltpu.sync_copy(dst_h.at[t, pl.ds(off, _CHUNK)], dstv)
                pltpu.async_copy(tabs_h.at[t].at[srcv], rows, sem).wait()
                pltpu.sync_copy(rows, acc.at[dstv], add=True)
                return carry

            lax.fori_loop(0, _NCHUNK, chunk, 0)
            plsc.subcore_barrier()
            pltpu.sync_copy(acc.at[pl.ds(rbase, _RPT)],
                            out_h.at[t, c, pl.ds(rbase, _RPT)])
            plsc.subcore_barrier()

    return body(tabs, src2, dst, zeros)


_TCR = 1000


def _tc_body(xw, xs, aWA, aWB, aSA, aSB,
             W1, b1, W2, b2, Wg1, bg1, Wg2, bg2, W3, b3, W4, b4,
             outw, outs):
    def matT(x, w):
        return lax.dot_general(x, w[:], (((1,), (1,)), ((), ())),
                               preferred_element_type=_f32)

    hw = matT(xw[:], W1) + b1[:][None, :]
    hs = matT(xs[:], W2) + b2[:][None, :]

    sumS = jnp.concatenate([aWA[0, :, :_HALF], aWB[0, :, :_HALF]], axis=1)
    degw = aWA[0, :, _HALF:_HALF + 1]
    tw = matT(matT(sumS, W2), Wg1)
    bias_w = matT(b2[:][None, :], Wg1) + bg1[:][None, :]
    nhw = (tw + degw * bias_w) / jnp.maximum(degw, 1.0)
    uw = matT(nhw + hw, W4) + b4[:][None, :]
    mw = jnp.max(uw, axis=1, keepdims=True)
    outw[:] = uw - (mw + jnp.log(jnp.sum(jnp.exp(uw - mw), axis=1,
                                         keepdims=True)))

    sumW = jnp.concatenate([aSA[0, :, :_HALF], aSB[0, :, :_HALF]], axis=1)
    degs = aSA[0, :, _HALF:_HALF + 1]
    ts = matT(matT(sumW, W1), Wg2)
    bias_s = matT(b1[:][None, :], Wg2) + bg2[:][None, :]
    nhs = (ts + degs * bias_s) / jnp.maximum(degs, 1.0)
    us = matT(nhs + hs, W3) + b3[:][None, :]
    ms = jnp.max(us, axis=1, keepdims=True)
    outs[:] = us - (ms + jnp.log(jnp.sum(jnp.exp(us - ms), axis=1,
                                         keepdims=True)))


def _tc_dense(Xw, Xs, Agg,
              W1, b1, W2, b2, Wg1, bg1, Wg2, bg2, W3, b3, W4, b4):
    grid = (_NW // _TCR,)
    row_spec = pl.BlockSpec((_TCR, _D), lambda i: (i, 0))
    w_spec = pl.BlockSpec((_D, _D), lambda i: (0, 0))
    b_spec = pl.BlockSpec((_D,), lambda i: (0,))

    def agg_spec(k):
        return pl.BlockSpec((1, _TCR, _FW), lambda i, k=k: (k, i, 0))

    Agg4 = Agg.reshape(4, _NW, _FW)
    return pl.pallas_call(
        _tc_body,
        grid=grid,
        in_specs=[row_spec, row_spec,
                  agg_spec(0), agg_spec(1), agg_spec(2), agg_spec(3),
                  w_spec, b_spec, w_spec, b_spec, w_spec, b_spec,
                  w_spec, b_spec, w_spec, b_spec, w_spec, b_spec],
        out_specs=[row_spec, row_spec],
        out_shape=[jax.ShapeDtypeStruct((_NW, _D), _f32),
                   jax.ShapeDtypeStruct((_NS, _D), _f32)],
    )(Xw, Xs, Agg4, Agg4, Agg4, Agg4,
      W1, b1, W2, b2, Wg1, bg1, Wg2, bg2, W3, b3, W4, b4)


def _make_table(X):
    ones = jnp.ones((X.shape[0], 1), _f32)
    pad = jnp.zeros((X.shape[0], _FW - _HALF - 1), _f32)
    lo = jnp.concatenate([X[:, :_HALF], ones, pad], axis=1)
    hi = jnp.concatenate([X[:, _HALF:], ones, pad], axis=1)
    return jnp.concatenate([lo, hi], axis=0)


def kernel(Xw, Xs, E, Erev, W1, b1, W2, b2, Wg1, bg1, Wg2, bg2, W3, b3, W4, b4):
    srcW = E[0].astype(jnp.int32)
    dstS = E[1].astype(jnp.int32)
    srcS = Erev[0].astype(jnp.int32)
    dstW = Erev[1].astype(jnp.int32)

    tabs = jnp.stack([_make_table(Xs), _make_table(Xw)])
    src2 = jnp.stack([jnp.stack([srcS, srcS + _NS]),
                      jnp.stack([srcW, srcW + _NW])])
    dst = jnp.stack([dstW, dstS])
    zeros = jnp.zeros((_RPT, _FW), _f32)

    Agg = _sc_aggregate(tabs, src2, dst, zeros)

    ow, os_ = _tc_dense(Xw, Xs, Agg,
                        W1, b1, W2, b2, Wg1, bg1, Wg2, bg2, W3, b3, W4, b4)
    return (ow, os_)


# speedup vs baseline: 4.1870x; 1.5503x over previous
import functools

import jax
import jax.numpy as jnp
from jax import lax
from jax.experimental import pallas as pl
from jax.experimental.pallas import tpu as pltpu
from jax.experimental.pallas import tpu_sc as plsc

_NW = 10000
_NS = 10000
_NE = 160000
_D = 256
_HALF = 128
_FW = 144
_NTILES = 16
_EPT = _NE // _NTILES
_CHUNK = 80
_NCHUNK = _EPT // _CHUNK
_RPT = _NW // _NTILES

_f32 = jnp.float32


def _sc_aggregate(tabs, src2, dst, zeros):
    mesh = plsc.VectorSubcoreMesh(core_axis_name="c", subcore_axis_name="s")

    @functools.partial(
        pl.kernel,
        mesh=mesh,
        out_type=jax.ShapeDtypeStruct((2, 2, _NW, _FW), _f32),
        scratch_types=[pltpu.VMEM((_NCHUNK, _CHUNK), jnp.int32),
                       pltpu.VMEM((2, _CHUNK), jnp.int32),
                       pltpu.VMEM((2, _CHUNK, _FW), _f32),
                       pltpu.VMEM_SHARED((_NW, _FW), _f32),
                       pltpu.SemaphoreType.DMA,
                       pltpu.SemaphoreType.DMA],
        compiler_params=pltpu.CompilerParams(use_tc_tiling_on_sc=False),
    )
    def body(tabs_h, src2_h, dst_h, zro_h, out_h, srcv, dstv, rows, acc,
             semg, semd):
        c = lax.axis_index("c")
        s = lax.axis_index("s")
        rbase = s * _RPT

        for t in range(2):
            pltpu.sync_copy(zro_h, acc.at[pl.ds(rbase, _RPT)])
            pltpu.sync_copy(src2_h.at[t, c, s], srcv)
            plsc.subcore_barrier()

            def start_chunk(j, buf):
                pltpu.async_copy(tabs_h.at[t].at[srcv.at[j]],
                                 rows.at[buf], semg)
                pltpu.async_copy(dst_h.at[t, s, j], dstv.at[buf], semd)

            def drain_chunk(buf):
                pltpu.make_async_copy(tabs_h.at[t, pl.ds(0, _CHUNK)],
                                      rows.at[buf], semg).wait()
                pltpu.make_async_copy(dst_h.at[t, s, 0],
                                      dstv.at[buf], semd).wait()

            def scatter(j, buf):
                pltpu.sync_copy(rows.at[buf], acc.at[dstv.at[buf]], add=True)

            start_chunk(0, 0)

            def pair(m, carry):
                j = 2 * m
                drain_chunk(0)
                start_chunk(j + 1, 1)
                scatter(j, 0)
                drain_chunk(1)
                start_chunk(j + 2, 0)
                scatter(j + 1, 1)
                return carry

            lax.fori_loop(0, (_NCHUNK - 1) // 2, pair, 0)
            drain_chunk(0)
            scatter(_NCHUNK - 1, 0)

            plsc.subcore_barrier()
            pltpu.sync_copy(acc.at[pl.ds(rbase, _RPT)],
                            out_h.at[t, c, pl.ds(rbase, _RPT)])
            plsc.subcore_barrier()

    return body(tabs, src2, dst, zeros)


_TCR = 1000


def _tc_body(xw, xs, aWA, aWB, aSA, aSB,
             W1, b1, W2, b2, Wg1, bg1, Wg2, bg2, W3, b3, W4, b4,
             outw, outs):
    def matT(x, w):
        return lax.dot_general(x, w[:], (((1,), (1,)), ((), ())),
                               preferred_element_type=_f32)

    hw = matT(xw[:], W1) + b1[:][None, :]
    hs = matT(xs[:], W2) + b2[:][None, :]

    sumS = jnp.concatenate([aWA[0, :, :_HALF], aWB[0, :, :_HALF]], axis=1)
    degw = aWA[0, :, _HALF:_HALF + 1]
    tw = matT(matT(sumS, W2), Wg1)
    bias_w = matT(b2[:][None, :], Wg1) + bg1[:][None, :]
    nhw = (tw + degw * bias_w) / jnp.maximum(degw, 1.0)
    uw = matT(nhw + hw, W4) + b4[:][None, :]
    mw = jnp.max(uw, axis=1, keepdims=True)
    outw[:] = uw - (mw + jnp.log(jnp.sum(jnp.exp(uw - mw), axis=1,
                                         keepdims=True)))

    sumW = jnp.concatenate([aSA[0, :, :_HALF], aSB[0, :, :_HALF]], axis=1)
    degs = aSA[0, :, _HALF:_HALF + 1]
    ts = matT(matT(sumW, W1), Wg2)
    bias_s = matT(b1[:][None, :], Wg2) + bg2[:][None, :]
    nhs = (ts + degs * bias_s) / jnp.maximum(degs, 1.0)
    us = matT(nhs + hs, W3) + b3[:][None, :]
    ms = jnp.max(us, axis=1, keepdims=True)
    outs[:] = us - (ms + jnp.log(jnp.sum(jnp.exp(us - ms), axis=1,
                                         keepdims=True)))


def _tc_dense(Xw, Xs, Agg,
              W1, b1, W2, b2, Wg1, bg1, Wg2, bg2, W3, b3, W4, b4):
    grid = (_NW // _TCR,)
    row_spec = pl.BlockSpec((_TCR, _D), lambda i: (i, 0))
    w_spec = pl.BlockSpec((_D, _D), lambda i: (0, 0))
    b_spec = pl.BlockSpec((_D,), lambda i: (0,))

    def agg_spec(k):
        return pl.BlockSpec((1, _TCR, _FW), lambda i, k=k: (k, i, 0))

    Agg4 = Agg.reshape(4, _NW, _FW)
    return pl.pallas_call(
        _tc_body,
        grid=grid,
        in_specs=[row_spec, row_spec,
                  agg_spec(0), agg_spec(1), agg_spec(2), agg_spec(3),
                  w_spec, b_spec, w_spec, b_spec, w_spec, b_spec,
                  w_spec, b_spec, w_spec, b_spec, w_spec, b_spec],
        out_specs=[row_spec, row_spec],
        out_shape=[jax.ShapeDtypeStruct((_NW, _D), _f32),
                   jax.ShapeDtypeStruct((_NS, _D), _f32)],
    )(Xw, Xs, Agg4, Agg4, Agg4, Agg4,
      W1, b1, W2, b2, Wg1, bg1, Wg2, bg2, W3, b3, W4, b4)


def _make_table(X):
    ones = jnp.ones((X.shape[0], 1), _f32)
    pad = jnp.zeros((X.shape[0], _FW - _HALF - 1), _f32)
    lo = jnp.concatenate([X[:, :_HALF], ones, pad], axis=1)
    hi = jnp.concatenate([X[:, _HALF:], ones, pad], axis=1)
    return jnp.concatenate([lo, hi], axis=0)


def kernel(Xw, Xs, E, Erev, W1, b1, W2, b2, Wg1, bg1, Wg2, bg2, W3, b3, W4, b4):
    srcW = E[0].astype(jnp.int32)
    dstS = E[1].astype(jnp.int32)
    srcS = Erev[0].astype(jnp.int32)
    dstW = Erev[1].astype(jnp.int32)

    tabs = jnp.stack([_make_table(Xs), _make_table(Xw)])
    src2 = jnp.stack([jnp.stack([srcS, srcS + _NS]),
                      jnp.stack([srcW, srcW + _NW])])
    src2 = src2.reshape(2, 2, _NTILES, _NCHUNK, _CHUNK)
    dst = jnp.stack([dstW, dstS])
    dst = dst.reshape(2, _NTILES, _NCHUNK, _CHUNK)
    zeros = jnp.zeros((_RPT, _FW), _f32)

    Agg = _sc_aggregate(tabs, src2, dst, zeros)

    ow, os_ = _tc_dense(Xw, Xs, Agg,
                        W1, b1, W2, b2, Wg1, bg1, Wg2, bg2, W3, b3, W4, b4)
    return (ow, os_)
